# baseline (speedup 1.0000x reference)
import jax
import jax.numpy as jnp
from jax import lax
from jax.experimental import pallas as pl
from jax.experimental.pallas import tpu as pltpu

_B, _N, _E = 4, 8192, 768
_M = 64
_TOP_K = 8
_EPS = 1e-06

_TB = 4096


def _router_body(wt_ref, bias_ref, x_ref, comb_ref, seq_out_ref):
    x = x_ref[0]
    seq_out_ref[0] = x
    logits = jnp.dot(x, wt_ref[...], preferred_element_type=jnp.float32)

    lt = logits.T
    m = jnp.max(lt, axis=0, keepdims=True)
    ex = jnp.exp(lt - m)
    probs = ex / jnp.sum(ex, axis=0, keepdims=True)
    adj = probs + bias_ref[...].T

    iota = lax.broadcasted_iota(jnp.int32, (_M, _TB), 0).astype(jnp.float32)
    work = adj
    idxs, ws = [], []
    for _ in range(_TOP_K):
        mx = jnp.max(work, axis=0, keepdims=True)
        ik = jnp.min(jnp.where(work == mx, iota, float(_M)), axis=0,
                     keepdims=True)
        hit = iota == ik
        wk = jnp.sum(jnp.where(hit, probs, 0.0), axis=0, keepdims=True)
        work = jnp.where(hit, -jnp.inf, work)
        idxs.append(ik)
        ws.append(wk)

    idx_t = jnp.concatenate(idxs, axis=0)
    w_t = jnp.concatenate(ws, axis=0)
    w_t = w_t / (jnp.sum(w_t, axis=0, keepdims=True) + _EPS)
    idx_f = lax.bitcast_convert_type(idx_t.T.astype(jnp.int32), jnp.float32)
    comb_ref[0] = jnp.concatenate(
        [logits, w_t.T, idx_f, jnp.zeros((_TB, 48), jnp.float32)], axis=-1
    )


@jax.jit
def _router(seq, wt, bias2d):
    npb = _N // _TB
    grid = (_B, npb)
    return pl.pallas_call(
        _router_body,
        grid=grid,
        compiler_params=pltpu.CompilerParams(vmem_limit_bytes=117440512),
        in_specs=[
            pl.BlockSpec((_E, _M), lambda b, i: (0, 0)),
            pl.BlockSpec((1, _M), lambda b, i: (0, 0)),
            pl.BlockSpec((1, _TB, _E), lambda b, i: (b, i, 0)),
        ],
        out_specs=[
            pl.BlockSpec((1, _TB, 128), lambda b, i: (b, i, 0)),
            pl.BlockSpec((1, _TB, _E), lambda b, i: (b, i, 0)),
        ],
        out_shape=[
            jax.ShapeDtypeStruct((_B, _N, 128), jnp.float32),
            jax.ShapeDtypeStruct((_B, _N, _E), jnp.float32),
        ],
    )(wt, bias2d, seq)


def kernel(seq, W, bias):
    wt = W.T
    bias2d = bias.reshape(1, _M)
    comb, seq_out = _router(seq, wt, bias2d)
    logits = comb[:, :, 0:_M]
    wv = comb[:, :, _M:_M + _TOP_K]
    idx = lax.bitcast_convert_type(
        comb[:, :, _M + _TOP_K:_M + 2 * _TOP_K], jnp.int32
    )
    return (logits, idx, seq_out, wv)


# speedup vs baseline: 1.1944x; 1.1944x over previous
import jax
import jax.numpy as jnp
from jax import lax
from jax.experimental import pallas as pl
from jax.experimental.pallas import tpu as pltpu

_B, _N, _E = 4, 8192, 768
_M = 64
_TOP_K = 8
_EPS = 1e-06

_TB = 4096


def _router_body(wt_ref, bias_ref, x_ref, logits_ref, idx_ref, w_ref, seq_out_ref):
    x = x_ref[0]
    seq_out_ref[0] = x
    logits = jnp.dot(x, wt_ref[...], preferred_element_type=jnp.float32)
    logits_ref[0] = logits

    lt = logits.T
    m = jnp.max(lt, axis=0, keepdims=True)
    ex = jnp.exp(lt - m)
    probs = ex / jnp.sum(ex, axis=0, keepdims=True)
    adj = probs + bias_ref[...].T

    iota = lax.broadcasted_iota(jnp.int32, (_M, _TB), 0).astype(jnp.float32)
    work = adj
    idxs, ws = [], []
    for _ in range(_TOP_K):
        mx = jnp.max(work, axis=0, keepdims=True)
        ik = jnp.min(jnp.where(work == mx, iota, float(_M)), axis=0,
                     keepdims=True)
        hit = iota == ik
        wk = jnp.sum(jnp.where(hit, probs, 0.0), axis=0, keepdims=True)
        work = jnp.where(hit, -jnp.inf, work)
        idxs.append(ik)
        ws.append(wk)

    idx_t = jnp.concatenate(idxs, axis=0)
    w_t = jnp.concatenate(ws, axis=0)
    w_t = w_t / (jnp.sum(w_t, axis=0, keepdims=True) + _EPS)
    idx_ref[0] = idx_t.T.astype(jnp.int32)
    w_ref[0] = w_t.T


@jax.jit
def _router(seq, wt, bias2d):
    npb = _N // _TB
    grid = (_B, npb)
    return pl.pallas_call(
        _router_body,
        grid=grid,
        compiler_params=pltpu.CompilerParams(vmem_limit_bytes=117440512),
        in_specs=[
            pl.BlockSpec((_E, _M), lambda b, i: (0, 0)),
            pl.BlockSpec((1, _M), lambda b, i: (0, 0)),
            pl.BlockSpec((1, _TB, _E), lambda b, i: (b, i, 0)),
        ],
        out_specs=[
            pl.BlockSpec((1, _TB, _M), lambda b, i: (b, i, 0)),
            pl.BlockSpec((1, _TB, _TOP_K), lambda b, i: (b, i, 0)),
            pl.BlockSpec((1, _TB, _TOP_K), lambda b, i: (b, i, 0)),
            pl.BlockSpec((1, _TB, _E), lambda b, i: (b, i, 0)),
        ],
        out_shape=[
            jax.ShapeDtypeStruct((_B, _N, _M), jnp.float32),
            jax.ShapeDtypeStruct((_B, _N, _TOP_K), jnp.int32),
            jax.ShapeDtypeStruct((_B, _N, _TOP_K), jnp.float32),
            jax.ShapeDtypeStruct((_B, _N, _E), jnp.float32),
        ],
    )(wt, bias2d, seq)


def kernel(seq, W, bias):
    wt = W.T
    bias2d = bias.reshape(1, _M)
    logits, idx, wv, seq_out = _router(seq, wt, bias2d)
    return (logits, idx, seq_out, wv)
